# Initial kernel scaffold; baseline (speedup 1.0000x reference)
import jax
import jax.numpy as jnp
from jax.experimental import pallas as pl


def kernel(x, edge_index, edge_attr, batch, W1, b1, W2, b2, W3, b3, Wl, bl):
    raise NotImplementedError("write your pallas kernel here")


# speedup vs baseline: 21.3940x; 21.3940x over previous
import functools

import jax
import jax.numpy as jnp
from jax import lax
from jax.experimental import pallas as pl
from jax.experimental.pallas import tpu as pltpu
from jax.experimental.pallas import tpu_sc as plsc

_N = 10000
_E = 320000
_D = 128
_H = 128
_C = 2
_G = 128

_NC = 2
_NS = 16
_NW = _NC * _NS
_K = 128
_NB = 79
_EW = _NB * _K
_EP = _NW * _EW

_CB = 16

_NBLK = 10
_ROWS = _N // _NBLK

_RPS = 632
_RPS_LAST = _N - 15 * _RPS


@functools.cache
def _make_sc_deg():
    return functools.partial(
        pl.kernel,
        out_type=jax.ShapeDtypeStruct((_NC * _N,), jnp.float32),
        mesh=plsc.VectorSubcoreMesh(core_axis_name="c", subcore_axis_name="s"),
        scratch_types=[
            pltpu.VMEM((_NB, _K), jnp.int32),
            pltpu.VMEM((_NB, _K), jnp.float32),
            pltpu.VMEM((1024,), jnp.float32),
            pltpu.VMEM_SHARED((_N,), jnp.float32),
        ],
    )(_sc_deg_body)


def _sc_deg_body(dst_hbm, w_hbm, out_hbm, dst_v, w_v, zb, deg):
    c = lax.axis_index("c")
    s = lax.axis_index("s")
    w = s * _NC + c

    @pl.loop(0, 64)
    def _zero(i):
        zb[pl.ds(i * 16, 16)] = jnp.zeros((16,), jnp.float32)

    @pl.when(s < 10)
    def _():
        pltpu.sync_copy(zb.at[pl.ds(0, 1000)], deg.at[pl.ds(s * 1000, 1000)])

    plsc.subcore_barrier()

    pltpu.sync_copy(dst_hbm.at[w], dst_v)
    pltpu.sync_copy(w_hbm.at[w], w_v)

    @pl.loop(0, _NB)
    def _scat(j):
        pltpu.sync_copy(w_v.at[j], deg.at[dst_v.at[j]], add=True)

    plsc.subcore_barrier()

    @pl.when(s < 10)
    def _():
        pltpu.sync_copy(deg.at[pl.ds(s * 1000, 1000)], zb.at[pl.ds(0, 1000)])
        pltpu.sync_copy(zb.at[pl.ds(0, 1000)],
                        out_hbm.at[pl.ds(c * _N + s * 1000, 1000)])


@functools.cache
def _make_sc_prop():
    return functools.partial(
        pl.kernel,
        out_type=jax.ShapeDtypeStruct((_NC * _N, _H), jnp.float32),
        mesh=plsc.VectorSubcoreMesh(core_axis_name="c", subcore_axis_name="s"),
        scratch_types=[
            pltpu.VMEM((_CB, _K), jnp.int32),
            pltpu.VMEM((_CB, _K), jnp.int32),
            pltpu.VMEM((_CB, _K), jnp.float32),
            pltpu.VMEM((_K, _H), jnp.float32),
            pltpu.VMEM((_K, _H), jnp.float32),
            pltpu.VMEM_SHARED((_N, _H), jnp.float32),
            pltpu.SemaphoreType.DMA,
            pltpu.SemaphoreType.DMA,
        ],
    )(_sc_prop_body)


def _sc_prop_body(hp_hbm, src_hbm, dst_hbm, w_hbm, out_hbm,
                  src_v, dst_v, w_v, bufa, bufb, acc, sema, semb):
    c = lax.axis_index("c")
    s = lax.axis_index("s")
    w = s * _NC + c

    @pl.loop(0, _K)
    def _zero(r):
        for d in range(_H // 16):
            bufa[r, pl.ds(d * 16, 16)] = jnp.zeros((16,), jnp.float32)

    base = s * _RPS
    for t in range(4):
        pltpu.sync_copy(bufa, acc.at[pl.ds(base + t * _K, _K)])

    @pl.when(s < 15)
    def _():
        pltpu.sync_copy(bufa.at[pl.ds(0, _RPS - 4 * _K)],
                        acc.at[pl.ds(base + 4 * _K, _RPS - 4 * _K)])

    @pl.when(s == 15)
    def _():
        pltpu.sync_copy(bufa.at[pl.ds(0, _RPS_LAST - 4 * _K)],
                        acc.at[pl.ds(base + 4 * _K, _RPS_LAST - 4 * _K)])

    plsc.subcore_barrier()

    def _scale(buf, jb):
        @pl.loop(0, _K // 16)
        def _(g):
            wv = w_v[jb, pl.ds(g * 16, 16)]
            for l in range(16):
                e = g * 16 + l
                ws = wv[l]
                for d in range(_H // 16):
                    sl = pl.ds(d * 16, 16)
                    buf[e, sl] = buf[e, sl] * ws

    for c0 in range(0, _NB, _CB):
        nb = min(_CB, _NB - c0)
        pltpu.sync_copy(src_hbm.at[w, pl.ds(c0, nb)], src_v.at[pl.ds(0, nb)])
        pltpu.sync_copy(dst_hbm.at[w, pl.ds(c0, nb)], dst_v.at[pl.ds(0, nb)])
        pltpu.sync_copy(w_hbm.at[w, pl.ds(c0, nb)], w_v.at[pl.ds(0, nb)])

        pltpu.async_copy(hp_hbm.at[src_v.at[0]], bufa, sema)

        @pl.loop(0, nb, step=2)
        def _main(j):
            @pl.when(j + 1 < nb)
            def _():
                pltpu.async_copy(hp_hbm.at[src_v.at[j + 1]], bufb, semb)

            pltpu.make_async_copy(hp_hbm.at[src_v.at[j]], bufa, sema).wait()
            _scale(bufa, j)
            pltpu.sync_copy(bufa, acc.at[dst_v.at[j]], add=True)

            @pl.when(j + 2 < nb)
            def _():
                pltpu.async_copy(hp_hbm.at[src_v.at[j + 2]], bufa, sema)

            @pl.when(j + 1 < nb)
            def _():
                pltpu.make_async_copy(hp_hbm.at[src_v.at[j + 1]], bufb,
                                      semb).wait()
                _scale(bufb, j + 1)
                pltpu.sync_copy(bufb, acc.at[dst_v.at[j + 1]], add=True)

    plsc.subcore_barrier()
    rbase = s * _RPS

    def _drain(tail):
        chunks = [(t * _K, _K) for t in range(4)] + [(4 * _K, tail)]
        for t, (off, ln) in enumerate(chunks):
            buf = bufa if t % 2 == 0 else bufb
            pltpu.sync_copy(acc.at[pl.ds(rbase + off, ln)],
                            buf.at[pl.ds(0, ln)])
            pltpu.sync_copy(buf.at[pl.ds(0, ln)],
                            out_hbm.at[pl.ds(c * _N + rbase + off, ln)])

    @pl.when(s < 15)
    def _():
        _drain(_RPS - 4 * _K)

    @pl.when(s == 15)
    def _():
        _drain(_RPS_LAST - 4 * _K)


def _tc1(x, W1, degt):
    def body(x_ref, w_ref, degt_ref, h1p_ref, dinv_ref):
        deg = degt_ref[0, 0, :] + degt_ref[0, 1, :] + 1.0
        dinv = jnp.where(deg > 0, lax.rsqrt(jnp.maximum(deg, 1e-12)), 0.0)
        dinv_ref[0, 0, :] = dinv
        h = jnp.dot(x_ref[...], w_ref[...], preferred_element_type=jnp.float32)
        h1p_ref[...] = h * dinv[:, None]

    return pl.pallas_call(
        body,
        grid=(_NBLK,),
        in_specs=[
            pl.BlockSpec((_ROWS, _D), lambda i: (i, 0)),
            pl.BlockSpec((_D, _H), lambda i: (0, 0)),
            pl.BlockSpec((1, 2, _ROWS), lambda i: (i, 0, 0)),
        ],
        out_specs=[
            pl.BlockSpec((_ROWS, _H), lambda i: (i, 0)),
            pl.BlockSpec((1, 1, _ROWS), lambda i: (i, 0, 0)),
        ],
        out_shape=[
            jax.ShapeDtypeStruct((_N, _H), jnp.float32),
            jax.ShapeDtypeStruct((_NBLK, 1, _ROWS), jnp.float32),
        ],
    )(x, W1, degt)


def _tc_mid(aggp, hp, dinv, b, W):
    def body(a_ref, hp_ref, dinv_ref, b_ref, w_ref, out_ref):
        a3 = a_ref[...]
        dv = dinv_ref[0, 0, :]
        t = (a3[0] + a3[1] + hp_ref[...]) * dv[:, None] + b_ref[...][None, :]
        xr = jnp.maximum(t, 0.0)
        out_ref[...] = jnp.dot(xr, w_ref[...],
                               preferred_element_type=jnp.float32) * dv[:, None]

    return pl.pallas_call(
        body,
        grid=(_NBLK,),
        in_specs=[
            pl.BlockSpec((2, _ROWS, _H), lambda i: (0, i, 0)),
            pl.BlockSpec((_ROWS, _H), lambda i: (i, 0)),
            pl.BlockSpec((1, 1, _ROWS), lambda i: (i, 0, 0)),
            pl.BlockSpec((_H,), lambda i: (0,)),
            pl.BlockSpec((_H, _H), lambda i: (0, 0)),
        ],
        out_specs=pl.BlockSpec((_ROWS, _H), lambda i: (i, 0)),
        out_shape=jax.ShapeDtypeStruct((_N, _H), jnp.float32),
    )(aggp, hp, dinv, b, W)


def _tc_fin(aggp, hp, dinv, b3, batch3, Wlp, blp):
    def body(a_ref, hp_ref, dinv_ref, b_ref, batch_ref, wl_ref, bl_ref,
             out_ref, accs, cnts):
        i = pl.program_id(0)
        a3 = a_ref[...]
        dv = dinv_ref[0, 0, :]
        h3 = (a3[0] + a3[1] + hp_ref[...]) * dv[:, None] + b_ref[...][None, :]
        bt = batch_ref[0, 0, :]
        oh = (bt[None, :] == lax.broadcasted_iota(jnp.int32, (_G, _ROWS), 0))
        oh = oh.astype(jnp.float32)
        ps = jnp.dot(oh, h3, preferred_element_type=jnp.float32)
        pc = jnp.sum(oh, axis=1)

        @pl.when(i == 0)
        def _():
            accs[...] = jnp.zeros_like(accs)
            cnts[...] = jnp.zeros_like(cnts)

        accs[...] += ps
        cnts[...] += pc

        @pl.when(i == _NBLK - 1)
        def _():
            pooled = accs[...] / jnp.maximum(cnts[...], 1.0)[:, None]
            out_ref[...] = jnp.dot(pooled, wl_ref[...],
                                   preferred_element_type=jnp.float32) \
                + bl_ref[...][None, :]

    return pl.pallas_call(
        body,
        grid=(_NBLK,),
        in_specs=[
            pl.BlockSpec((2, _ROWS, _H), lambda i: (0, i, 0)),
            pl.BlockSpec((_ROWS, _H), lambda i: (i, 0)),
            pl.BlockSpec((1, 1, _ROWS), lambda i: (i, 0, 0)),
            pl.BlockSpec((_H,), lambda i: (0,)),
            pl.BlockSpec((1, 1, _ROWS), lambda i: (i, 0, 0)),
            pl.BlockSpec((_H, 128), lambda i: (0, 0)),
            pl.BlockSpec((128,), lambda i: (0,)),
        ],
        out_specs=pl.BlockSpec((_G, 128), lambda i: (0, 0)),
        out_shape=jax.ShapeDtypeStruct((_G, 128), jnp.float32),
        scratch_shapes=[
            pltpu.VMEM((_G, 128), jnp.float32),
            pltpu.VMEM((_G,), jnp.float32),
        ],
    )(aggp, hp, dinv, b3, batch3, Wlp, blp)


def kernel(x, edge_index, edge_attr, batch, W1, b1, W2, b2, W3, b3, Wl, bl):
    src = edge_index[0]
    dst = edge_index[1]
    npad = _EP - _E
    padidx = (jnp.arange(npad, dtype=jnp.int32) % _N)
    srcp = jnp.concatenate([src, padidx]).reshape(_NW, _NB, _K)
    dstp = jnp.concatenate([dst, padidx]).reshape(_NW, _NB, _K)
    wp = jnp.concatenate(
        [edge_attr, jnp.zeros((npad,), jnp.float32)]).reshape(_NW, _NB, _K)

    degp = _make_sc_deg()(dstp, wp)
    degt = degp.reshape(2, _NBLK, _ROWS).transpose(1, 0, 2)
    h1p, dinv = _tc1(x, W1, degt)

    Ws = jnp.stack([W2, W3, W2])
    bs = jnp.stack([b1, b2, b3])

    def _step(carry, wb):
        _, hp, _ = carry
        W_i, b_i = wb
        agg = _make_sc_prop()(hp, srcp, dstp, wp).reshape(2, _N, _H)
        hp_next = _tc_mid(agg, hp, dinv, b_i, W_i)
        return (hp, hp_next, agg), None

    agg0 = jnp.zeros((2, _N, _H), jnp.float32)
    (h3p, _, agg3), _ = lax.scan(_step, (h1p, h1p, agg0), (Ws, bs))

    Wlp = jnp.zeros((_H, 128), jnp.float32).at[:, :_C].set(Wl)
    blp = jnp.zeros((128,), jnp.float32).at[:_C].set(bl)
    batch3 = batch.reshape(_NBLK, 1, _ROWS)
    outp = _tc_fin(agg3, h3p, dinv, b3, batch3, Wlp, blp)
    return outp[:, :_C]
